# Initial kernel scaffold; baseline (speedup 1.0000x reference)
import jax
import jax.numpy as jnp
from jax.experimental import pallas as pl


def kernel(z_e_x, codebook):
    raise NotImplementedError("write your pallas kernel here")


# speedup vs baseline: 1.0889x; 1.0889x over previous
import jax
import jax.numpy as jnp
from jax.experimental import pallas as pl

_K = 8192
_CHUNK = 4096
_BM = 512


def _vq_body(lhs_ref, z_ref, cb_ref, out_ref):
    lhs = lhs_ref[...]
    z = z_ref[...]
    zn = jnp.sum(z * z, axis=1, keepdims=True)
    cur_val = jnp.full((_BM,), jnp.inf, dtype=jnp.float32)
    cur_idx = jnp.zeros((_BM,), dtype=jnp.int32)
    for t in range(_K // _CHUNK):
        cb = cb_ref[pl.ds(t * _CHUNK, _CHUNK), :]
        cn = jnp.sum(cb * cb, axis=1)
        conv = jax.lax.dot_general(
            lhs, cb, (((1,), (1,)), ((), ())),
            preferred_element_type=jnp.float32,
        )
        d = (zn - conv) + cn[None, :]
        tmin = jnp.min(d, axis=1)
        ids = jax.lax.broadcasted_iota(jnp.int32, (_BM, _CHUNK), 1)
        targ = jnp.min(jnp.where(d == tmin[:, None], ids, _K), axis=1)
        upd = tmin < cur_val
        cur_idx = jnp.where(upd, targ + t * _CHUNK, cur_idx)
        cur_val = jnp.where(
            upd, tmin.astype(jnp.bfloat16).astype(jnp.float32), cur_val)
    out_ref[...] = cur_idx.reshape(1, 1, _BM)


def kernel(z_e_x, codebook):
    B, D, T = z_e_x.shape
    z = jnp.transpose(z_e_x, (0, 2, 1)).reshape(B * T, D)
    lhs = (2.0 * z).astype(jnp.bfloat16)
    M = B * T
    nb = M // _BM
    out = pl.pallas_call(
        _vq_body,
        grid=(nb,),
        in_specs=[
            pl.BlockSpec((_BM, D), lambda m: (m, 0)),
            pl.BlockSpec((_BM, D), lambda m: (m, 0)),
            pl.BlockSpec((_K, D), lambda m: (0, 0)),
        ],
        out_specs=pl.BlockSpec((1, 1, _BM), lambda m: (m, 0, 0)),
        out_shape=jax.ShapeDtypeStruct((nb, 1, _BM), jnp.int32),
    )(lhs, z, codebook)
    return out.reshape(B, T)


# speedup vs baseline: 1.2564x; 1.1538x over previous
import jax
import jax.numpy as jnp
from jax.experimental import pallas as pl

_K = 8192
_CHUNK = 4096
_BM = 512


def _vq_body(lhs_ref, z_ref, cb_ref, ids_ref, out_ref):
    lhs = lhs_ref[...]
    z = z_ref[...]
    ids = ids_ref[...]
    zn = jnp.sum(z * z, axis=1, keepdims=True)
    cur_val = jnp.full((_BM,), jnp.inf, dtype=jnp.float32)
    cur_idx = jnp.zeros((_BM,), dtype=jnp.float32)
    for t in range(_K // _CHUNK):
        cb = cb_ref[pl.ds(t * _CHUNK, _CHUNK), :]
        cn = jnp.sum(cb * cb, axis=1)
        conv = jax.lax.dot_general(
            lhs, cb, (((1,), (1,)), ((), ())),
            preferred_element_type=jnp.float32,
        )
        d = (zn - conv) + cn[None, :]
        tmin = jnp.min(d, axis=1)
        big = jnp.float32(_K)
        d3 = d.reshape(_BM // 8, 8, _CHUNK)
        cand = jnp.where(d3 == tmin.reshape(_BM // 8, 8, 1),
                         ids[None, :, :], big)
        targ = jnp.min(cand, axis=2).reshape(_BM)
        upd = tmin < cur_val
        cur_idx = jnp.where(upd, targ + jnp.float32(t * _CHUNK), cur_idx)
        cur_val = jnp.where(
            upd, tmin.astype(jnp.bfloat16).astype(jnp.float32), cur_val)
    out_ref[...] = cur_idx.astype(jnp.int32).reshape(1, 1, _BM)


def kernel(z_e_x, codebook):
    B, D, T = z_e_x.shape
    z = jnp.transpose(z_e_x, (0, 2, 1)).reshape(B * T, D)
    lhs = (2.0 * z).astype(jnp.bfloat16)
    ids = jnp.broadcast_to(
        jnp.arange(_CHUNK, dtype=jnp.float32)[None, :], (8, _CHUNK))
    M = B * T
    nb = M // _BM
    out = pl.pallas_call(
        _vq_body,
        grid=(nb,),
        in_specs=[
            pl.BlockSpec((_BM, D), lambda m: (m, 0)),
            pl.BlockSpec((_BM, D), lambda m: (m, 0)),
            pl.BlockSpec((_K, D), lambda m: (0, 0)),
            pl.BlockSpec((8, _CHUNK), lambda m: (0, 0)),
        ],
        out_specs=pl.BlockSpec((1, 1, _BM), lambda m: (m, 0, 0)),
        out_shape=jax.ShapeDtypeStruct((nb, 1, _BM), jnp.int32),
    )(lhs, z, codebook, ids)
    return out.reshape(B, T)


# speedup vs baseline: 1.3308x; 1.0593x over previous
import jax
import jax.numpy as jnp
from jax.experimental import pallas as pl

_K = 8192
_CHUNK = 4096
_BM = 1024


def _vq_body(lhs_ref, z_ref, cb_ref, ids_ref, out_ref):
    lhs = lhs_ref[...]
    z = z_ref[...]
    ids = ids_ref[...]
    zn = jnp.sum(z * z, axis=1, keepdims=True)
    cur_val = jnp.full((_BM,), jnp.inf, dtype=jnp.float32)
    cur_idx = jnp.zeros((_BM,), dtype=jnp.float32)
    for t in range(_K // _CHUNK):
        cb = cb_ref[pl.ds(t * _CHUNK, _CHUNK), :]
        cn = jnp.sum(cb * cb, axis=1)
        conv = jax.lax.dot_general(
            lhs, cb, (((1,), (1,)), ((), ())),
            preferred_element_type=jnp.float32,
        )
        d = (zn - conv) + cn[None, :]
        tmin = jnp.min(d, axis=1)
        big = jnp.float32(_K)
        d3 = d.reshape(_BM // 8, 8, _CHUNK)
        cand = jnp.where(d3 == tmin.reshape(_BM // 8, 8, 1),
                         ids[None, :, :], big)
        targ = jnp.min(cand, axis=2).reshape(_BM)
        upd = tmin < cur_val
        cur_idx = jnp.where(upd, targ + jnp.float32(t * _CHUNK), cur_idx)
        cur_val = jnp.where(
            upd, tmin.astype(jnp.bfloat16).astype(jnp.float32), cur_val)
    out_ref[...] = cur_idx.astype(jnp.int32).reshape(1, 1, _BM)


def kernel(z_e_x, codebook):
    B, D, T = z_e_x.shape
    z = jnp.transpose(z_e_x, (0, 2, 1)).reshape(B * T, D)
    lhs = (2.0 * z).astype(jnp.bfloat16)
    ids = jnp.broadcast_to(
        jnp.arange(_CHUNK, dtype=jnp.float32)[None, :], (8, _CHUNK))
    M = B * T
    nb = M // _BM
    out = pl.pallas_call(
        _vq_body,
        grid=(nb,),
        in_specs=[
            pl.BlockSpec((_BM, D), lambda m: (m, 0)),
            pl.BlockSpec((_BM, D), lambda m: (m, 0)),
            pl.BlockSpec((_K, D), lambda m: (0, 0)),
            pl.BlockSpec((8, _CHUNK), lambda m: (0, 0)),
        ],
        out_specs=pl.BlockSpec((1, 1, _BM), lambda m: (m, 0, 0)),
        out_shape=jax.ShapeDtypeStruct((nb, 1, _BM), jnp.int32),
    )(lhs, z, codebook, ids)
    return out.reshape(B, T)


# speedup vs baseline: 1.3565x; 1.0193x over previous
import jax
import jax.numpy as jnp
from jax.experimental import pallas as pl

_K = 8192
_CHUNK = 4096
_BM = 2048


def _vq_body(lhs_ref, z_ref, cb_ref, ids_ref, out_ref):
    lhs = lhs_ref[...]
    z = z_ref[...]
    ids = ids_ref[...]
    zn = jnp.sum(z * z, axis=1, keepdims=True)
    cur_val = jnp.full((_BM,), jnp.inf, dtype=jnp.float32)
    cur_idx = jnp.zeros((_BM,), dtype=jnp.float32)
    for t in range(_K // _CHUNK):
        cb = cb_ref[pl.ds(t * _CHUNK, _CHUNK), :]
        cn = jnp.sum(cb * cb, axis=1)
        conv = jax.lax.dot_general(
            lhs, cb, (((1,), (1,)), ((), ())),
            preferred_element_type=jnp.float32,
        )
        d = (zn - conv) + cn[None, :]
        tmin = jnp.min(d, axis=1)
        big = jnp.float32(_K)
        d3 = d.reshape(_BM // 8, 8, _CHUNK)
        cand = jnp.where(d3 == tmin.reshape(_BM // 8, 8, 1),
                         ids[None, :, :], big)
        targ = jnp.min(cand, axis=2).reshape(_BM)
        upd = tmin < cur_val
        cur_idx = jnp.where(upd, targ + jnp.float32(t * _CHUNK), cur_idx)
        cur_val = jnp.where(
            upd, tmin.astype(jnp.bfloat16).astype(jnp.float32), cur_val)
    out_ref[...] = cur_idx.astype(jnp.int32).reshape(1, 1, _BM)


def kernel(z_e_x, codebook):
    B, D, T = z_e_x.shape
    z = jnp.transpose(z_e_x, (0, 2, 1)).reshape(B * T, D)
    lhs = (2.0 * z).astype(jnp.bfloat16)
    ids = jnp.broadcast_to(
        jnp.arange(_CHUNK, dtype=jnp.float32)[None, :], (8, _CHUNK))
    M = B * T
    nb = M // _BM
    out = pl.pallas_call(
        _vq_body,
        grid=(nb,),
        in_specs=[
            pl.BlockSpec((_BM, D), lambda m: (m, 0)),
            pl.BlockSpec((_BM, D), lambda m: (m, 0)),
            pl.BlockSpec((_K, D), lambda m: (0, 0)),
            pl.BlockSpec((8, _CHUNK), lambda m: (0, 0)),
        ],
        out_specs=pl.BlockSpec((1, 1, _BM), lambda m: (m, 0, 0)),
        out_shape=jax.ShapeDtypeStruct((nb, 1, _BM), jnp.int32),
    )(lhs, z, codebook, ids)
    return out.reshape(B, T)


# speedup vs baseline: 1.5167x; 1.1181x over previous
import jax
import jax.numpy as jnp
from jax.experimental import pallas as pl

_K = 8192
_CHUNK = 4096
_G = 128
_BM = 2048


def _vq_body(lhs_ref, z_ref, cb_ref, ids_ref, out_ref):
    lhs = lhs_ref[...]
    z = z_ref[...]
    ids8 = ids_ref[...]
    zn = jnp.sum(z * z, axis=1, keepdims=True)
    big = jnp.float32(_K)
    cur_val = jnp.full((_BM,), jnp.inf, dtype=jnp.float32)
    cur_idx = jnp.zeros((_BM,), dtype=jnp.float32)
    for t in range(_K // _CHUNK):
        cb = cb_ref[pl.ds(t * _CHUNK, _CHUNK), :]
        cn = jnp.sum(cb * cb, axis=1)
        conv = jax.lax.dot_general(
            lhs, cb, (((1,), (1,)), ((), ())),
            preferred_element_type=jnp.float32,
        )
        mval = jnp.full((_BM // 8, 8, _G), jnp.inf, dtype=jnp.float32)
        midx = jnp.zeros((_BM // 8, 8, _G), dtype=jnp.float32)
        for g in range(_CHUNK // _G):
            dg = (zn - conv[:, g * _G:(g + 1) * _G]) + cn[None, g * _G:(g + 1) * _G]
            d3 = dg.reshape(_BM // 8, 8, _G)
            kg = (ids8 + jnp.float32(g * _G))[None, :, :]
            upd = d3 < mval
            midx = jnp.where(upd, kg, midx)
            mval = jnp.minimum(mval, d3)
        tmin = jnp.min(mval.reshape(_BM, _G), axis=1)
        cand = jnp.where(mval == tmin.reshape(_BM // 8, 8, 1), midx, big)
        targ = jnp.min(cand, axis=2).reshape(_BM)
        upd_c = tmin < cur_val
        cur_idx = jnp.where(upd_c, targ + jnp.float32(t * _CHUNK), cur_idx)
        cur_val = jnp.where(
            upd_c, tmin.astype(jnp.bfloat16).astype(jnp.float32), cur_val)
    out_ref[...] = cur_idx.astype(jnp.int32).reshape(1, 1, _BM)


def kernel(z_e_x, codebook):
    B, D, T = z_e_x.shape
    z = jnp.transpose(z_e_x, (0, 2, 1)).reshape(B * T, D)
    lhs = (2.0 * z).astype(jnp.bfloat16)
    ids = jnp.broadcast_to(
        jnp.arange(_G, dtype=jnp.float32)[None, :], (8, _G))
    M = B * T
    nb = M // _BM
    out = pl.pallas_call(
        _vq_body,
        grid=(nb,),
        in_specs=[
            pl.BlockSpec((_BM, D), lambda m: (m, 0)),
            pl.BlockSpec((_BM, D), lambda m: (m, 0)),
            pl.BlockSpec((_K, D), lambda m: (0, 0)),
            pl.BlockSpec((8, _G), lambda m: (0, 0)),
        ],
        out_specs=pl.BlockSpec((1, 1, _BM), lambda m: (m, 0, 0)),
        out_shape=jax.ShapeDtypeStruct((nb, 1, _BM), jnp.int32),
    )(lhs, z, codebook, ids)
    return out.reshape(B, T)
